# Initial kernel scaffold; baseline (speedup 1.0000x reference)
import jax
import jax.numpy as jnp
from jax.experimental import pallas as pl


def kernel(feature, feature_geo, xyz, center, plane_centers, plane_normals, plane_xyz_min, plane_xyz_max, W1, b1, W2, b2, W3, b3):
    raise NotImplementedError("write your pallas kernel here")


# speedup vs baseline: 3.7325x; 3.7325x over previous
import jax
import jax.numpy as jnp
from jax.experimental import pallas as pl
from jax.experimental.pallas import tpu as pltpu

N = 50000
D = 128
P = 64
BN = 2000


def _fused_kernel(feat_ref, geo_ref, cl8_ref, prm_ref,
                  w1_ref, w2_ref, w3_ref, b1_ref, b2_ref, b3_ref,
                  pl_ref, on_ref, off_ref, acc_on, acc_off):
    i = pl.program_id(0)
    nblocks = pl.num_programs(0)

    @pl.when(i == 0)
    def _init():
        acc_on[...] = jnp.zeros_like(acc_on)
        acc_off[...] = jnp.zeros_like(acc_off)

    c8 = cl8_ref[...]
    prm = prm_ref[...]
    x = c8[:, 0:1]
    y = c8[:, 1:2]
    z = c8[:, 2:3]
    rmask = ((x >= prm[0:1, :]) & (x < prm[1:2, :]) &
             (y >= prm[2:3, :]) & (y < prm[3:4, :]))
    bf = lambda a: a.astype(jnp.bfloat16).astype(jnp.float32)
    dx = bf(x - prm[4:5, :])
    dy = bf(y - prm[5:6, :])
    dz = bf(z - prm[6:7, :])
    dist = jnp.abs(dx * bf(prm[7:8, :]) + dy * bf(prm[8:9, :])
                   + dz * bf(prm[9:10, :]))
    mask = rmask & (dist < 0.1)

    fcat = jnp.concatenate([feat_ref[...], geo_ref[...]], axis=1)
    h1 = jnp.maximum(
        jnp.dot(fcat, w1_ref[...], preferred_element_type=jnp.float32)
        + b1_ref[...], 0.0)
    h2 = jnp.maximum(
        jnp.dot(h1, w2_ref[...], preferred_element_type=jnp.float32)
        + b2_ref[...], 0.0)
    logit = (jnp.dot(h2, w3_ref[...], preferred_element_type=jnp.float32)
             + b3_ref[...])

    pl_ref[...] = jnp.where(mask, logit, 0.0)

    on_m = mask & (logit > 0.0)
    off_m = mask & (logit <= 0.0)

    on_rows = []
    off_rows = []
    for p in range(P):
        onc = on_m[:, p:p + 1]
        offc = off_m[:, p:p + 1]
        on_rows.append(jnp.max(jnp.where(onc, h2, 0.0), axis=0, keepdims=True))
        off_rows.append(jnp.max(jnp.where(offc, h2, 0.0), axis=0, keepdims=True))
    acc_on[...] = jnp.maximum(acc_on[...], jnp.concatenate(on_rows, axis=0))
    acc_off[...] = jnp.maximum(acc_off[...], jnp.concatenate(off_rows, axis=0))

    @pl.when(i == nblocks - 1)
    def _write():
        on_ref[...] = acc_on[...]
        off_ref[...] = acc_off[...]


def kernel(feature, feature_geo, xyz, center, plane_centers, plane_normals,
           plane_xyz_min, plane_xyz_max, W1, b1, W2, b2, W3, b3):
    f32 = jnp.float32
    clouds = xyz + center
    cl8 = jnp.zeros((N, 8), f32).at[:, :3].set(clouds)
    prm = jnp.zeros((16, P), f32)
    prm = prm.at[0, :].set(plane_xyz_min[:, 0])
    prm = prm.at[1, :].set(plane_xyz_max[:, 0])
    prm = prm.at[2, :].set(plane_xyz_min[:, 1])
    prm = prm.at[3, :].set(plane_xyz_max[:, 1])
    prm = prm.at[4:7, :].set(plane_centers.T)
    prm = prm.at[7:10, :].set(plane_normals.T)

    grid = (N // BN,)
    full = lambda a: pl.BlockSpec(a.shape, lambda i: (0,) * a.ndim)

    b1r = b1.reshape(1, D)
    b2r = b2.reshape(1, D)
    b3r = b3.reshape(1, 1)

    pl_nt, on_f, off_f = pl.pallas_call(
        _fused_kernel,
        grid=grid,
        in_specs=[
            pl.BlockSpec((BN, D), lambda i: (i, 0)),
            pl.BlockSpec((BN, D), lambda i: (i, 0)),
            pl.BlockSpec((BN, 8), lambda i: (i, 0)),
            full(prm),
            full(W1), full(W2), full(W3),
            full(b1r), full(b2r), full(b3r),
        ],
        out_specs=[
            pl.BlockSpec((BN, P), lambda i: (i, 0)),
            pl.BlockSpec((P, D), lambda i: (0, 0)),
            pl.BlockSpec((P, D), lambda i: (0, 0)),
        ],
        out_shape=[
            jax.ShapeDtypeStruct((N, P), f32),
            jax.ShapeDtypeStruct((P, D), f32),
            jax.ShapeDtypeStruct((P, D), f32),
        ],
        scratch_shapes=[
            pltpu.VMEM((P, D), f32),
            pltpu.VMEM((P, D), f32),
        ],
        compiler_params=pltpu.CompilerParams(
            dimension_semantics=("arbitrary",),
        ),
    )(feature, feature_geo, cl8, prm, W1, W2, W3, b1r, b2r, b3r)

    return jnp.concatenate(
        [pl_nt.T.reshape(-1), on_f.reshape(-1), off_f.reshape(-1)])


# speedup vs baseline: 6.7359x; 1.8047x over previous
import jax
import jax.numpy as jnp
from jax.experimental import pallas as pl
from jax.experimental.pallas import tpu as pltpu

N = 50000
D = 128
P = 64
BN = 2000


def _fused_kernel(feat_ref, geo_ref, cl8_ref, prm_ref,
                  w1_ref, w2_ref, w3_ref, b1_ref, b2_ref, b3_ref,
                  pl_ref, on_ref, off_ref, acc_on, acc_off):
    i = pl.program_id(0)
    nblocks = pl.num_programs(0)

    @pl.when(i == 0)
    def _init():
        acc_on[...] = jnp.zeros_like(acc_on)
        acc_off[...] = jnp.zeros_like(acc_off)

    c8 = cl8_ref[...]
    prm = prm_ref[...]
    x = c8[:, 0:1]
    y = c8[:, 1:2]
    z = c8[:, 2:3]
    rmask = ((x >= prm[0:1, :]) & (x < prm[1:2, :]) &
             (y >= prm[2:3, :]) & (y < prm[3:4, :]))
    bf = lambda a: a.astype(jnp.bfloat16).astype(jnp.float32)
    dx = bf(x - prm[4:5, :])
    dy = bf(y - prm[5:6, :])
    dz = bf(z - prm[6:7, :])
    dist = jnp.abs(dx * bf(prm[7:8, :]) + dy * bf(prm[8:9, :])
                   + dz * bf(prm[9:10, :]))
    mask = rmask & (dist < 0.1)

    fcat = jnp.concatenate([feat_ref[...], geo_ref[...]], axis=1)
    h1 = jnp.maximum(
        jnp.dot(fcat, w1_ref[...], preferred_element_type=jnp.float32)
        + b1_ref[...], 0.0)
    h2 = jnp.maximum(
        jnp.dot(h1, w2_ref[...], preferred_element_type=jnp.float32)
        + b2_ref[...], 0.0)
    logit = (jnp.dot(h2, w3_ref[...], preferred_element_type=jnp.float32)
             + b3_ref[...])

    pl_ref[...] = jnp.where(mask, logit, 0.0)

    h2_on = jnp.where(logit > 0.0, h2, 0.0)
    h2_off = h2 - h2_on

    on_rows = []
    off_rows = []
    for p in range(P):
        mcol = mask[:, p:p + 1]
        on_rows.append(jnp.max(jnp.where(mcol, h2_on, 0.0), axis=0, keepdims=True))
        off_rows.append(jnp.max(jnp.where(mcol, h2_off, 0.0), axis=0, keepdims=True))
    acc_on[...] = jnp.maximum(acc_on[...], jnp.concatenate(on_rows, axis=0))
    acc_off[...] = jnp.maximum(acc_off[...], jnp.concatenate(off_rows, axis=0))

    @pl.when(i == nblocks - 1)
    def _write():
        on_ref[...] = acc_on[...]
        off_ref[...] = acc_off[...]


def kernel(feature, feature_geo, xyz, center, plane_centers, plane_normals,
           plane_xyz_min, plane_xyz_max, W1, b1, W2, b2, W3, b3):
    f32 = jnp.float32
    clouds = xyz + center
    cl8 = jnp.zeros((N, 8), f32).at[:, :3].set(clouds)
    prm = jnp.zeros((16, P), f32)
    prm = prm.at[0, :].set(plane_xyz_min[:, 0])
    prm = prm.at[1, :].set(plane_xyz_max[:, 0])
    prm = prm.at[2, :].set(plane_xyz_min[:, 1])
    prm = prm.at[3, :].set(plane_xyz_max[:, 1])
    prm = prm.at[4:7, :].set(plane_centers.T)
    prm = prm.at[7:10, :].set(plane_normals.T)

    grid = (N // BN,)
    full = lambda a: pl.BlockSpec(a.shape, lambda i: (0,) * a.ndim)

    b1r = b1.reshape(1, D)
    b2r = b2.reshape(1, D)
    b3r = b3.reshape(1, 1)

    pl_nt, on_f, off_f = pl.pallas_call(
        _fused_kernel,
        grid=grid,
        in_specs=[
            pl.BlockSpec((BN, D), lambda i: (i, 0)),
            pl.BlockSpec((BN, D), lambda i: (i, 0)),
            pl.BlockSpec((BN, 8), lambda i: (i, 0)),
            full(prm),
            full(W1), full(W2), full(W3),
            full(b1r), full(b2r), full(b3r),
        ],
        out_specs=[
            pl.BlockSpec((BN, P), lambda i: (i, 0)),
            pl.BlockSpec((P, D), lambda i: (0, 0)),
            pl.BlockSpec((P, D), lambda i: (0, 0)),
        ],
        out_shape=[
            jax.ShapeDtypeStruct((N, P), f32),
            jax.ShapeDtypeStruct((P, D), f32),
            jax.ShapeDtypeStruct((P, D), f32),
        ],
        scratch_shapes=[
            pltpu.VMEM((P, D), f32),
            pltpu.VMEM((P, D), f32),
        ],
        compiler_params=pltpu.CompilerParams(
            dimension_semantics=("arbitrary",),
        ),
    )(feature, feature_geo, cl8, prm, W1, W2, W3, b1r, b2r, b3r)

    return jnp.concatenate(
        [pl_nt.T.reshape(-1), on_f.reshape(-1), off_f.reshape(-1)])
